# baseline (speedup 1.0000x reference)
import jax
import jax.numpy as jnp
from jax.experimental import pallas as pl
from jax.experimental.pallas import tpu as pltpu

_TAU = 0.9
_TOP_P = 0.8
_TB = 1024


def _router_body(p_ref, w_ref, cb_ref, fcw_ref, fcb_ref, o_ref, acc_ref):
    h = pl.program_id(1)
    w = w_ref[...]
    cb = cb_ref[...]

    x = p_ref[:, 0, 0, :]
    hc = jax.lax.dot_general(
        w, x, (((1,), (0,)), ((), ())),
        preferred_element_type=jnp.float32)
    s8 = jnp.maximum(hc + cb, 0.0)

    @pl.when(h == 0)
    def _init():
        acc_ref[...] = s8

    @pl.when(h > 0)
    def _acc():
        acc_ref[...] = acc_ref[...] + s8

    @pl.when(h == pl.num_programs(1) - 1)
    def _finish():
        pooled = acc_ref[...] * (1.0 / 64.0)
        logits = jax.lax.dot_general(
            fcw_ref[...], pooled, (((1,), (0,)), ((), ())),
            preferred_element_type=jnp.float32) + fcb_ref[...]

        li = logits * (1.0 / _TAU)
        li = li - jnp.max(li, axis=0, keepdims=True)
        e = jnp.exp(li)
        probs = e / jnp.sum(e, axis=0, keepdims=True)

        for v in range(_TB // 128):
            p = probs[:, v * 128:(v + 1) * 128]
            pi = p[:, None, :]
            pj = p[None, :, :]
            ii = jax.lax.broadcasted_iota(jnp.int32, (16, 16, 128), 0)
            jj = jax.lax.broadcasted_iota(jnp.int32, (16, 16, 128), 1)
            g = (pj > pi) | ((pj == pi) & (jj <= ii))
            s = jnp.sum(jnp.where(g, jnp.broadcast_to(pj, (16, 16, 128)), 0.0),
                        axis=1)
            cnt = jnp.sum(g.astype(jnp.int32), axis=1)
            keep = (s <= _TOP_P) | (cnt < 2)
            masked = jnp.where(keep, p, 0.0)
            denom = jnp.clip(jnp.sum(masked, axis=0, keepdims=True),
                             1e-10, None)
            o_ref[:, v * 128:(v + 1) * 128] = masked / denom


def kernel(patch, conv_w, conv_b, fc_w, fc_b, layer_idx, threshold):
    del layer_idx, threshold
    n_tok = patch.shape[0]
    q = patch.transpose(1, 2, 3, 0).reshape(196, 64, 1, n_tok)

    grid = (n_tok // _TB, 64)
    out = pl.pallas_call(
        _router_body,
        grid=grid,
        in_specs=[
            pl.BlockSpec((196, 1, 1, _TB), lambda tb, h: (0, h, 0, tb)),
            pl.BlockSpec((128, 196), lambda tb, h: (0, 0)),
            pl.BlockSpec((128, 1), lambda tb, h: (0, 0)),
            pl.BlockSpec((16, 128), lambda tb, h: (0, 0)),
            pl.BlockSpec((16, 1), lambda tb, h: (0, 0)),
        ],
        out_specs=pl.BlockSpec((16, _TB), lambda tb, h: (0, tb)),
        out_shape=jax.ShapeDtypeStruct((16, n_tok), jnp.float32),
        scratch_shapes=[pltpu.VMEM((128, _TB), jnp.float32)],
    )(q, conv_w, conv_b.reshape(128, 1), fc_w, fc_b.reshape(16, 1))
    return out.T


# speedup vs baseline: 7.5425x; 7.5425x over previous
import jax
import jax.numpy as jnp
from jax.experimental import pallas as pl
from jax.experimental.pallas import tpu as pltpu

_TAU = 0.9
_TOP_P = 0.8
_TB = 2048


def _router_body(p_ref, w_ref, cb_ref, fcw_ref, fcb_ref, o_ref, acc_ref):
    h = pl.program_id(1)
    w = w_ref[...]
    cb = cb_ref[...]

    parts = []
    for v in range(8):
        x = p_ref[:, v, :]
        hc = jax.lax.dot_general(
            w, x, (((1,), (0,)), ((), ())),
            preferred_element_type=jnp.float32)
        parts.append(jnp.maximum(hc + cb, 0.0))
    s8 = ((parts[0] + parts[1]) + (parts[2] + parts[3])) + \
         ((parts[4] + parts[5]) + (parts[6] + parts[7]))

    @pl.when(h == 0)
    def _init():
        acc_ref[...] = s8

    @pl.when(h > 0)
    def _acc():
        acc_ref[...] = acc_ref[...] + s8

    @pl.when(h == pl.num_programs(1) - 1)
    def _finish():
        pooled = acc_ref[...] * (1.0 / 64.0)
        logits = jax.lax.dot_general(
            fcw_ref[...], pooled, (((1,), (0,)), ((), ())),
            preferred_element_type=jnp.float32) + fcb_ref[...]

        li = logits * (1.0 / _TAU)
        li = li - jnp.max(li, axis=0, keepdims=True)
        e = jnp.exp(li)
        probs = e / jnp.sum(e, axis=0, keepdims=True)

        for v in range(_TB // 128):
            p = probs[:, v * 128:(v + 1) * 128]
            pi = p[:, None, :]
            pj = p[None, :, :]
            ii = jax.lax.broadcasted_iota(jnp.int32, (16, 16, 128), 0)
            jj = jax.lax.broadcasted_iota(jnp.int32, (16, 16, 128), 1)
            g = (pj > pi) | ((pj == pi) & (jj <= ii))
            s = jnp.sum(jnp.where(g, jnp.broadcast_to(pj, (16, 16, 128)), 0.0),
                        axis=1)
            cnt = jnp.sum(g.astype(jnp.int32), axis=1)
            keep = (s <= _TOP_P) | (cnt < 2)
            masked = jnp.where(keep, p, 0.0)
            denom = jnp.clip(jnp.sum(masked, axis=0, keepdims=True),
                             1e-10, None)
            o_ref[:, v * 128:(v + 1) * 128] = masked / denom


def kernel(patch, conv_w, conv_b, fc_w, fc_b, layer_idx, threshold):
    del layer_idx, threshold
    n_tok = patch.shape[0]
    q = patch.transpose(1, 2, 3, 0).reshape(196, 64, n_tok)

    grid = (n_tok // _TB, 8)
    out = pl.pallas_call(
        _router_body,
        grid=grid,
        in_specs=[
            pl.BlockSpec((196, 8, _TB), lambda tb, h: (0, h, tb)),
            pl.BlockSpec((128, 196), lambda tb, h: (0, 0)),
            pl.BlockSpec((128, 1), lambda tb, h: (0, 0)),
            pl.BlockSpec((16, 128), lambda tb, h: (0, 0)),
            pl.BlockSpec((16, 1), lambda tb, h: (0, 0)),
        ],
        out_specs=pl.BlockSpec((16, _TB), lambda tb, h: (0, tb)),
        out_shape=jax.ShapeDtypeStruct((16, n_tok), jnp.float32),
        scratch_shapes=[pltpu.VMEM((128, _TB), jnp.float32)],
    )(q, conv_w, conv_b.reshape(128, 1), fc_w, fc_b.reshape(16, 1))
    return out.T
